# baseline (speedup 1.0000x reference)
import functools

import jax
import jax.numpy as jnp
from jax import lax
from jax.experimental import pallas as pl
from jax.experimental.pallas import tpu as pltpu
from jax.experimental.pallas import tpu_sc as plsc

_NC, _NS, _LANES = 2, 16, 16
_NW = _NC * _NS
_SUB = 16
_NBUF = 2
_B_SC = 1
_TC_SEQ_BLOCK = 256


def _sc_part(x, pe_table):
    batch, seq_len, d_model = x.shape
    b0 = batch - _B_SC
    s_per_w = seq_len // _NW
    n_sub = s_per_w // _SUB
    vecs = _SUB * d_model // _LANES

    mesh = plsc.VectorSubcoreMesh(
        core_axis_name="c", subcore_axis_name="s",
        num_cores=_NC, num_subcores=_NS)

    @functools.partial(
        pl.kernel,
        mesh=mesh,
        out_type=jax.ShapeDtypeStruct((_B_SC, seq_len, d_model), jnp.float32),
        scratch_types=[
            pltpu.VMEM((_NBUF, _B_SC, _SUB, d_model), jnp.float32),
            pltpu.VMEM((_NBUF, _B_SC, _SUB, d_model), jnp.float32),
            pltpu.SemaphoreType.DMA,
            pltpu.SemaphoreType.DMA,
            pltpu.SemaphoreType.DMA,
            pltpu.SemaphoreType.DMA,
        ],
        compiler_params=pltpu.CompilerParams(use_tc_tiling_on_sc=True),
    )
    def sc_add(x_hbm, pe_hbm, out_hbm, pe_v, x_v, si0, si1, so0, so1):
        wid = lax.axis_index("s") * _NC + lax.axis_index("c")
        w_s0 = wid * s_per_w
        sin = (si0, si1)
        sout = (so0, so1)

        def start_in(par, c):
            s0 = w_s0 + c * _SUB
            pltpu.async_copy(pe_hbm.at[pl.ds(s0, _SUB), :],
                             pe_v.at[par, 0], sin[par])
            pltpu.async_copy(x_hbm.at[pl.ds(b0, _B_SC), pl.ds(s0, _SUB), :],
                             x_v.at[par], sin[par])

        def wait_in(par):
            pltpu.make_async_copy(
                pe_hbm.at[pl.ds(w_s0, _SUB), :], pe_v.at[par, 0],
                sin[par]).wait()
            pltpu.make_async_copy(
                x_hbm.at[pl.ds(b0, _B_SC), pl.ds(w_s0, _SUB), :],
                x_v.at[par], sin[par]).wait()

        def start_out(par, c):
            s0 = w_s0 + c * _SUB
            pltpu.async_copy(x_v.at[par],
                             out_hbm.at[:, pl.ds(s0, _SUB), :], sout[par])

        def wait_out(par):
            pltpu.make_async_copy(
                x_v.at[par], out_hbm.at[:, pl.ds(w_s0, _SUB), :],
                sout[par]).wait()

        def compute(par):
            @plsc.parallel_loop(0, vecs, unroll=4)
            def _(i):
                r = lax.shift_right_logical(i, 6)
                off = pl.multiple_of(lax.mul(lax.rem(i, 64), _LANES), _LANES)
                pv = pe_v[par, 0, r, pl.ds(off, _LANES)]
                for j in range(_B_SC):
                    x_v[par, j, r, pl.ds(off, _LANES)] = (
                        x_v[par, j, r, pl.ds(off, _LANES)] + pv)

        for par in range(_NBUF):
            start_in(par, par)

        def body(k, _):
            for par in range(_NBUF):
                c = _NBUF * k + par
                wait_in(par)
                compute(par)
                start_out(par, c)
            for par in range(_NBUF):
                c_next = _NBUF * (k + 1) + par

                @pl.when(c_next < n_sub)
                def _():
                    wait_out(par)
                    start_in(par, c_next)

            return 0

        lax.fori_loop(0, n_sub // _NBUF, body, 0)
        for par in range(_NBUF):
            wait_out(par)

    return sc_add(x, pe_table)


def _tc_body(x_ref, pe_ref, o_ref):
    o_ref[...] = x_ref[...] + pe_ref[...][None, :, :]


def _tc_part(x, pe_table):
    batch, seq_len, d_model = x.shape
    b_tc = batch - _B_SC
    grid = (seq_len // _TC_SEQ_BLOCK,)
    return pl.pallas_call(
        _tc_body,
        grid=grid,
        in_specs=[
            pl.BlockSpec((b_tc, _TC_SEQ_BLOCK, d_model), lambda i: (0, i, 0)),
            pl.BlockSpec((_TC_SEQ_BLOCK, d_model), lambda i: (i, 0)),
        ],
        out_specs=pl.BlockSpec((b_tc, _TC_SEQ_BLOCK, d_model),
                               lambda i: (0, i, 0)),
        out_shape=jax.ShapeDtypeStruct((b_tc, seq_len, d_model), x.dtype),
    )(x, pe_table)


def kernel(x, pe_table):
    tc_out = _tc_part(x, pe_table)
    sc_out = _sc_part(x, pe_table)
    return jnp.concatenate([tc_out, sc_out], axis=0)


# speedup vs baseline: 1.3058x; 1.3058x over previous
import functools

import jax
import jax.numpy as jnp
from jax import lax
from jax.experimental import pallas as pl
from jax.experimental.pallas import tpu as pltpu
from jax.experimental.pallas import tpu_sc as plsc

_NC, _NS, _LANES = 2, 16, 16
_NW = _NC * _NS
_SUB = 4
_NBUF = 4


def kernel(x, pe_table):
    batch, seq_len, d_model = x.shape
    s_per_w = seq_len // _NW
    n_sub = s_per_w // _SUB
    vecs = _SUB * d_model // _LANES

    mesh = plsc.VectorSubcoreMesh(
        core_axis_name="c", subcore_axis_name="s",
        num_cores=_NC, num_subcores=_NS)

    @functools.partial(
        pl.kernel,
        mesh=mesh,
        out_type=jax.ShapeDtypeStruct((batch, seq_len, d_model), jnp.float32),
        scratch_types=[
            pltpu.VMEM((_NBUF, _SUB, d_model), jnp.float32),
            pltpu.VMEM((_NBUF, batch, _SUB, d_model), jnp.float32),
        ] + [pltpu.SemaphoreType.DMA] * (2 * _NBUF),
        compiler_params=pltpu.CompilerParams(use_tc_tiling_on_sc=True),
    )
    def sc_add(x_hbm, pe_hbm, out_hbm, pe_v, x_v, *sems):
        wid = lax.axis_index("s") * _NC + lax.axis_index("c")
        w_s0 = wid * s_per_w
        sin = sems[:_NBUF]
        sout = sems[_NBUF:]

        def start_in(par, c):
            s0 = w_s0 + c * _SUB
            pltpu.async_copy(pe_hbm.at[pl.ds(s0, _SUB), :], pe_v.at[par],
                             sin[par])
            pltpu.async_copy(x_hbm.at[:, pl.ds(s0, _SUB), :], x_v.at[par],
                             sin[par])

        def wait_in(par):
            pltpu.make_async_copy(
                pe_hbm.at[pl.ds(w_s0, _SUB), :], pe_v.at[par], sin[par]).wait()
            pltpu.make_async_copy(
                x_hbm.at[:, pl.ds(w_s0, _SUB), :], x_v.at[par], sin[par]).wait()

        def start_out(par, c):
            s0 = w_s0 + c * _SUB
            pltpu.async_copy(x_v.at[par], out_hbm.at[:, pl.ds(s0, _SUB), :],
                             sout[par])

        def wait_out(par):
            pltpu.make_async_copy(
                x_v.at[par], out_hbm.at[:, pl.ds(w_s0, _SUB), :],
                sout[par]).wait()

        def compute(par):
            @plsc.parallel_loop(0, vecs, unroll=4)
            def _(i):
                r = lax.shift_right_logical(i, 6)
                off = pl.multiple_of(
                    lax.mul(lax.rem(i, 64), _LANES), _LANES)
                pv = pe_v[par, r, pl.ds(off, _LANES)]
                for b in range(batch):
                    x_v[par, b, r, pl.ds(off, _LANES)] = (
                        x_v[par, b, r, pl.ds(off, _LANES)] + pv)

        for par in range(_NBUF):
            start_in(par, par)

        def body(k, _):
            for par in range(_NBUF):
                c = _NBUF * k + par
                wait_in(par)
                compute(par)
                start_out(par, c)
            for par in range(_NBUF):
                c_next = _NBUF * (k + 1) + par

                @pl.when(c_next < n_sub)
                def _():
                    wait_out(par)
                    start_in(par, c_next)

            return 0

        lax.fori_loop(0, n_sub // _NBUF, body, 0)
        for par in range(_NBUF):
            wait_out(par)

    return sc_add(x, pe_table)


# speedup vs baseline: 1.4159x; 1.0843x over previous
import functools

import jax
import jax.numpy as jnp
from jax import lax
from jax.experimental import pallas as pl
from jax.experimental.pallas import tpu as pltpu
from jax.experimental.pallas import tpu_sc as plsc

_NC, _NS, _LANES = 2, 16, 16
_NW = _NC * _NS
_SUB = 4
_NBUF = 2


def kernel(x, pe_table):
    batch, seq_len, d_model = x.shape
    s_per_w = seq_len // _NW
    n_sub = s_per_w // _SUB
    vecs = _SUB * d_model // _LANES

    mesh = plsc.VectorSubcoreMesh(
        core_axis_name="c", subcore_axis_name="s",
        num_cores=_NC, num_subcores=_NS)

    @functools.partial(
        pl.kernel,
        mesh=mesh,
        out_type=jax.ShapeDtypeStruct((batch, seq_len, d_model), jnp.float32),
        scratch_types=[
            pltpu.VMEM((_NBUF, _SUB, d_model), jnp.float32),
            pltpu.VMEM((_NBUF, batch, _SUB, d_model), jnp.float32),
            pltpu.VMEM((_NBUF, batch, _SUB, d_model), jnp.float32),
        ] + [pltpu.SemaphoreType.DMA] * (2 * _NBUF),
        compiler_params=pltpu.CompilerParams(use_tc_tiling_on_sc=True),
    )
    def sc_add(x_hbm, pe_hbm, out_hbm, pe_v, xi_v, xo_v, *sems):
        wid = lax.axis_index("s") * _NC + lax.axis_index("c")
        w_s0 = wid * s_per_w
        sin = sems[:_NBUF]
        sout = sems[_NBUF:]

        def start_in(par, c):
            s0 = w_s0 + c * _SUB
            pltpu.async_copy(pe_hbm.at[pl.ds(s0, _SUB), :], pe_v.at[par],
                             sin[par])
            pltpu.async_copy(x_hbm.at[:, pl.ds(s0, _SUB), :], xi_v.at[par],
                             sin[par])

        def wait_in(par):
            pltpu.make_async_copy(
                pe_hbm.at[pl.ds(w_s0, _SUB), :], pe_v.at[par], sin[par]).wait()
            pltpu.make_async_copy(
                x_hbm.at[:, pl.ds(w_s0, _SUB), :], xi_v.at[par], sin[par]).wait()

        def start_out(par, c):
            s0 = w_s0 + c * _SUB
            pltpu.async_copy(xo_v.at[par], out_hbm.at[:, pl.ds(s0, _SUB), :],
                             sout[par])

        def wait_out(par):
            pltpu.make_async_copy(
                xo_v.at[par], out_hbm.at[:, pl.ds(w_s0, _SUB), :],
                sout[par]).wait()

        def compute(par):
            @plsc.parallel_loop(0, vecs, unroll=4)
            def _(i):
                r = lax.shift_right_logical(i, 6)
                off = pl.multiple_of(lax.mul(lax.rem(i, 64), _LANES), _LANES)
                pv = pe_v[par, r, pl.ds(off, _LANES)]
                for b in range(batch):
                    xo_v[par, b, r, pl.ds(off, _LANES)] = (
                        xi_v[par, b, r, pl.ds(off, _LANES)] + pv)

        for par in range(_NBUF):
            start_in(par, par)

        def body(k, _):
            for par in range(_NBUF):
                c = _NBUF * k + par
                wait_in(par)

                @pl.when(c >= _NBUF)
                def _():
                    wait_out(par)

                compute(par)
                start_out(par, c)
                c_next = c + _NBUF

                @pl.when(c_next < n_sub)
                def _():
                    start_in(par, c_next)

            return 0

        lax.fori_loop(0, n_sub // _NBUF, body, 0)
        for par in range(_NBUF):
            wait_out(par)

    return sc_add(x, pe_table)
